# baseline (speedup 1.0000x reference)
import functools

import jax
import jax.numpy as jnp
from jax.experimental import pallas as pl
from jax.experimental.pallas import tpu as pltpu


def _gap_sublane_kernel(x_ref, o_ref, *, inv_hw):
    s = jnp.sum(x_ref[...], axis=1, dtype=jnp.float32)
    o_ref[...] = (s * inv_hw).astype(o_ref.dtype)


@jax.jit
def kernel(x):
    N, C, H, W = x.shape
    hw = H * W
    inv_hw = 1.0 / float(hw)

    y = x.transpose(0, 2, 3, 1).reshape(N, hw, C)

    bn = 32 if N % 32 == 0 else (8 if N % 8 == 0 else 1)
    num_tiles = N // bn

    out = pl.pallas_call(
        functools.partial(_gap_sublane_kernel, inv_hw=inv_hw),
        out_shape=jax.ShapeDtypeStruct((N, C), x.dtype),
        grid_spec=pl.GridSpec(
            grid=(num_tiles,),
            in_specs=[pl.BlockSpec((bn, hw, C), lambda i: (i, 0, 0))],
            out_specs=pl.BlockSpec((bn, C), lambda i: (i, 0)),
        ),
        compiler_params=pltpu.CompilerParams(
            dimension_semantics=("parallel",),
            vmem_limit_bytes=64 * 1024 * 1024,
        ),
        cost_estimate=pl.CostEstimate(
            flops=N * C * H * W,
            bytes_accessed=N * C * H * W * x.dtype.itemsize
            + N * C * x.dtype.itemsize,
            transcendentals=0,
        ),
    )(y)
    return out


# speedup vs baseline: 1.0363x; 1.0363x over previous
import functools

import jax
import jax.numpy as jnp
from jax.experimental import pallas as pl
from jax.experimental.pallas import tpu as pltpu


def _gap_sublane_kernel(x_ref, o_ref, *, inv_hw):
    s = jnp.sum(x_ref[...], axis=1, dtype=jnp.float32)
    o_ref[...] = (s * inv_hw).astype(o_ref.dtype)


@jax.jit
def kernel(x):
    N, C, H, W = x.shape
    hw = H * W
    inv_hw = 1.0 / float(hw)

    y = x.transpose(0, 2, 3, 1).reshape(N, hw, C)

    bn = 16 if N % 16 == 0 else (8 if N % 8 == 0 else 1)
    num_tiles = N // bn

    out = pl.pallas_call(
        functools.partial(_gap_sublane_kernel, inv_hw=inv_hw),
        out_shape=jax.ShapeDtypeStruct((N, C), x.dtype),
        grid_spec=pl.GridSpec(
            grid=(num_tiles,),
            in_specs=[pl.BlockSpec((bn, hw, C), lambda i: (i, 0, 0))],
            out_specs=pl.BlockSpec((bn, C), lambda i: (i, 0)),
        ),
        compiler_params=pltpu.CompilerParams(
            dimension_semantics=("parallel",),
            vmem_limit_bytes=64 * 1024 * 1024,
        ),
        cost_estimate=pl.CostEstimate(
            flops=N * C * H * W,
            bytes_accessed=N * C * H * W * x.dtype.itemsize
            + N * C * x.dtype.itemsize,
            transcendentals=0,
        ),
    )(y)
    return out
